# Initial kernel scaffold; baseline (speedup 1.0000x reference)
import jax
import jax.numpy as jnp
from jax.experimental import pallas as pl


def kernel(distance, labels):
    raise NotImplementedError("write your pallas kernel here")


# speedup vs baseline: 1.0314x; 1.0314x over previous
import functools

import jax
import jax.numpy as jnp
from jax import lax
from jax.experimental import pallas as pl
from jax.experimental.pallas import tpu as pltpu
from jax.experimental.pallas import tpu_sc as plsc

_B = 4096
_C = 100000
_L = 16
_TILES = 16
_PER_TILE = _B // _TILES
_HALF = _PER_TILE // 2


def _sc_body(dist_hbm, labels_hbm, out_hbm,
             lab_v, idx_a, idx_b, vals_a, vals_b, acc_v, shared, tot_v, sem):
    sid = lax.axis_index("s")
    base = sid * _PER_TILE

    pltpu.sync_copy(labels_hbm.at[pl.ds(base, _PER_TILE)], lab_v)

    for j in range(_PER_TILE // _L):
        lab = lab_v[pl.ds(j * _L, _L)]
        i_vec = base + j * _L + lax.iota(jnp.int32, _L)
        flat = i_vec * _C + lab
        if j < _HALF // _L:
            idx_a[pl.ds(j * _L, _L)] = flat
        else:
            idx_b[pl.ds((j - _HALF // _L) * _L, _L)] = flat

    cp_a = pltpu.async_copy(dist_hbm.at[idx_a], vals_a, sem)
    cp_b = pltpu.async_copy(dist_hbm.at[idx_b], vals_b, sem)
    cp_a.wait()
    cp_b.wait()

    acc = jnp.zeros((_L,), jnp.float32)
    for j in range(_HALF // _L):
        acc = acc + vals_a[pl.ds(j * _L, _L)] + vals_b[pl.ds(j * _L, _L)]
    acc_v[...] = acc

    pltpu.sync_copy(acc_v, shared.at[pl.ds(pl.multiple_of(sid * _L, 8), _L)])
    plsc.subcore_barrier()

    @pl.when(sid == 0)
    def _():
        pltpu.sync_copy(shared, tot_v)
        tot = jnp.zeros((_L,), jnp.float32)
        for s in range(_TILES):
            tot = tot + tot_v[pl.ds(s * _L, _L)]
        iota = lax.iota(jnp.int32, _L)
        dnums = lax.GatherDimensionNumbers(
            offset_dims=(), collapsed_slice_dims=(0,), start_index_map=(0,))
        for shift in (8, 4, 2, 1):
            perm = lax.bitwise_xor(iota, shift)
            tot = tot + lax.gather(
                tot, perm[:, None], dnums, (1,),
                mode=lax.GatherScatterMode.PROMISE_IN_BOUNDS)
        acc_v[...] = tot * jnp.float32(1.0 / _B)
        pltpu.sync_copy(acc_v, out_hbm)


@jax.jit
def _sc_loss(dist_flat, labels):
    mesh = plsc.VectorSubcoreMesh(core_axis_name="c", subcore_axis_name="s",
                                  num_cores=1)
    run = functools.partial(
        pl.kernel, mesh=mesh,
        out_type=jax.ShapeDtypeStruct((_L,), jnp.float32),
        scratch_types=[
            pltpu.VMEM((_PER_TILE,), jnp.int32),
            pltpu.VMEM((_HALF,), jnp.int32),
            pltpu.VMEM((_HALF,), jnp.int32),
            pltpu.VMEM((_HALF,), jnp.float32),
            pltpu.VMEM((_HALF,), jnp.float32),
            pltpu.VMEM((_L,), jnp.float32),
            pltpu.VMEM_SHARED((_TILES * _L,), jnp.float32),
            pltpu.VMEM((_TILES * _L,), jnp.float32),
            pltpu.SemaphoreType.DMA,
        ],
    )(_sc_body)
    return run(dist_flat, labels)


def kernel(distance, labels):
    labels = labels.astype(jnp.int32)
    dist_flat = distance.reshape(-1)
    out = _sc_loss(dist_flat, labels)
    return out[0]


# speedup vs baseline: 48.7005x; 47.2178x over previous
import functools

import jax
import jax.numpy as jnp
from jax import lax
from jax.experimental import pallas as pl
from jax.experimental.pallas import tpu as pltpu
from jax.experimental.pallas import tpu_sc as plsc

_B = 4096
_C = 100000
_L = 16
_TILES = 16
_PER = _B // _TILES
_G = 16
_NCH = _PER // _G


def _sc_body(distT_hbm, labels_hbm, out_hbm,
             lab_v, idx_v, rows_v, acc_v, shared, tot_v, sem):
    sid = lax.axis_index("s")
    base = sid * _PER

    pltpu.sync_copy(labels_hbm.at[pl.ds(base, _PER)], lab_v)

    acc = jnp.zeros((_L,), jnp.float32)
    iota = lax.iota(jnp.int32, _L)
    for ch in range(_NCH):
        idx_v[...] = lab_v[pl.ds(ch * _G, _G)]
        pltpu.async_copy(distT_hbm.at[idx_v], rows_v, sem).wait()
        for t_loc in range(_G):
            t = ch * _G + t_loc
            v = rows_v[t_loc,
                       pl.ds(pl.multiple_of(base + (t // _L) * _L, 8), _L)]
            acc = acc + jnp.where(iota == (t % _L), v, 0.0)
    acc_v[...] = acc

    pltpu.sync_copy(acc_v, shared.at[pl.ds(pl.multiple_of(sid * _L, 8), _L)])
    plsc.subcore_barrier()

    @pl.when(sid == 0)
    def _():
        pltpu.sync_copy(shared, tot_v)
        tot = jnp.zeros((_L,), jnp.float32)
        for s in range(_TILES):
            tot = tot + tot_v[pl.ds(s * _L, _L)]
        dnums = lax.GatherDimensionNumbers(
            offset_dims=(), collapsed_slice_dims=(0,), start_index_map=(0,))
        for shift in (8, 4, 2, 1):
            perm = lax.bitwise_xor(iota, shift)
            tot = tot + lax.gather(
                tot, perm[:, None], dnums, (1,),
                mode=lax.GatherScatterMode.PROMISE_IN_BOUNDS)
        acc_v[...] = tot * jnp.float32(1.0 / _B)
        pltpu.sync_copy(acc_v, out_hbm)


@jax.jit
def _sc_loss(distT, labels):
    mesh = plsc.VectorSubcoreMesh(core_axis_name="c", subcore_axis_name="s",
                                  num_cores=1)
    run = functools.partial(
        pl.kernel, mesh=mesh,
        out_type=jax.ShapeDtypeStruct((_L,), jnp.float32),
        scratch_types=[
            pltpu.VMEM((_PER,), jnp.int32),
            pltpu.VMEM((_G,), jnp.int32),
            pltpu.VMEM((_G, _B), jnp.float32),
            pltpu.VMEM((_L,), jnp.float32),
            pltpu.VMEM_SHARED((_TILES * _L,), jnp.float32),
            pltpu.VMEM((_TILES * _L,), jnp.float32),
            pltpu.SemaphoreType.DMA,
        ],
    )(_sc_body)
    return run(distT, labels)


def kernel(distance, labels):
    labels = labels.astype(jnp.int32)
    out = _sc_loss(distance.T, labels)
    return out[0]


# speedup vs baseline: 66.1392x; 1.3581x over previous
import functools

import jax
import jax.numpy as jnp
from jax import lax
from jax.experimental import pallas as pl
from jax.experimental.pallas import tpu as pltpu
from jax.experimental.pallas import tpu_sc as plsc

_B = 4096
_C = 100000
_L = 16
_CORES = 2
_TILES = 16
_PER = _B // (_CORES * _TILES)
_G = 16
_NCH = _PER // _G


def _sc_body(distT_hbm, labels_hbm, out_hbm,
             lab_v, idx_v, rows_v, acc_v, shared, tot_v, sem):
    cid = lax.axis_index("c")
    sid = lax.axis_index("s")
    base = (cid * _TILES + sid) * _PER

    pltpu.sync_copy(labels_hbm.at[pl.ds(pl.multiple_of(base, 8), _PER)], lab_v)

    acc = jnp.zeros((_L,), jnp.float32)
    iota = lax.iota(jnp.int32, _L)
    for ch in range(_NCH):
        idx_v[...] = lab_v[pl.ds(ch * _G, _G)]
        pltpu.async_copy(distT_hbm.at[idx_v], rows_v, sem).wait()
        for t_loc in range(_G):
            t = ch * _G + t_loc
            v = rows_v[t_loc,
                       pl.ds(pl.multiple_of(base + (t // _L) * _L, 8), _L)]
            acc = acc + jnp.where(iota == (t % _L), v, 0.0)
    acc_v[...] = acc

    pltpu.sync_copy(acc_v, shared.at[pl.ds(pl.multiple_of(sid * _L, 8), _L)])
    plsc.subcore_barrier()

    @pl.when(sid == 0)
    def _():
        pltpu.sync_copy(shared, tot_v)
        tot = jnp.zeros((_L,), jnp.float32)
        for s in range(_TILES):
            tot = tot + tot_v[pl.ds(s * _L, _L)]
        dnums = lax.GatherDimensionNumbers(
            offset_dims=(), collapsed_slice_dims=(0,), start_index_map=(0,))
        for shift in (8, 4, 2, 1):
            perm = lax.bitwise_xor(iota, shift)
            tot = tot + lax.gather(
                tot, perm[:, None], dnums, (1,),
                mode=lax.GatherScatterMode.PROMISE_IN_BOUNDS)
        acc_v[...] = tot * jnp.float32(1.0 / _B)
        pltpu.sync_copy(acc_v,
                        out_hbm.at[pl.ds(pl.multiple_of(cid * _L, 8), _L)])


@jax.jit
def _sc_loss(distT, labels):
    mesh = plsc.VectorSubcoreMesh(core_axis_name="c", subcore_axis_name="s",
                                  num_cores=_CORES)
    run = functools.partial(
        pl.kernel, mesh=mesh,
        out_type=jax.ShapeDtypeStruct((_CORES * _L,), jnp.float32),
        scratch_types=[
            pltpu.VMEM((_PER,), jnp.int32),
            pltpu.VMEM((_G,), jnp.int32),
            pltpu.VMEM((_G, _B), jnp.float32),
            pltpu.VMEM((_L,), jnp.float32),
            pltpu.VMEM_SHARED((_TILES * _L,), jnp.float32),
            pltpu.VMEM((_TILES * _L,), jnp.float32),
            pltpu.SemaphoreType.DMA,
        ],
    )(_sc_body)
    return run(distT, labels)


def kernel(distance, labels):
    labels = labels.astype(jnp.int32)
    out = _sc_loss(distance.T, labels)
    return out[0] + out[_L]
